# baseline (speedup 1.0000x reference)
import functools

import jax
import jax.numpy as jnp
from jax import lax
from jax.experimental import pallas as pl
from jax.experimental.pallas import tpu as pltpu
from jax.experimental.pallas import tpu_sc as plsc

B, T, D = 16, 2048, 128
NH = 8
NPROJ = 32
BUCKET = 64
NBUCKET = T // BUCKET
SCALE = D ** -0.5


def _hash_body(x_ref, proj_ref, out_ref):
    x = x_ref[0]
    p = proj_ref[...]
    px = lax.dot_general(x, p, (((1,), (0,)), ((), ())),
                         preferred_element_type=jnp.float32)
    px3 = px.reshape(T, NH, NPROJ)
    m = jnp.max(jnp.abs(px3), axis=-1, keepdims=True)
    iota = lax.broadcasted_iota(jnp.int32, (T, NH, NPROJ), 2)
    big = jnp.int32(2 * NPROJ)
    cand = jnp.minimum(jnp.where(px3 >= m, iota, big),
                       jnp.where(px3 <= -m, iota + NPROJ, big))
    out_ref[0] = jnp.min(cand, axis=-1)


def _hash_call(x2, proj2):
    return pl.pallas_call(
        _hash_body,
        grid=(2 * B,),
        in_specs=[
            pl.BlockSpec((1, T, D), lambda i: (i, 0, 0)),
            pl.BlockSpec((D, NH * NPROJ), lambda i: (0, 0)),
        ],
        out_specs=pl.BlockSpec((1, T, NH), lambda i: (i, 0, 0)),
        out_shape=jax.ShapeDtypeStruct((2 * B, T, NH), jnp.int32),
    )(x2, proj2)


def _bk_slice(ref, n):
    lo = (n - 1) % NBUCKET
    if n == 0:
        return jnp.concatenate(
            [ref[0, pl.ds(lo * BUCKET, BUCKET), :],
             ref[0, pl.ds(0, BUCKET), :]], axis=0)
    return ref[0, pl.ds(lo * BUCKET, 2 * BUCKET), :]


def _attn_body(sq_ref, sk_ref, sv_ref, so_ref, lse_ref, dots_ref):
    for n in range(NBUCKET):
        bq = sq_ref[0, pl.ds(n * BUCKET, BUCKET), :]
        dots_ref[pl.ds(n * BUCKET, BUCKET), :] = lax.dot_general(
            bq, _bk_slice(sk_ref, n), (((1,), (1,)), ((), ())),
            preferred_element_type=jnp.float32)
    d = dots_ref[...] * SCALE
    m = jnp.max(d, axis=-1, keepdims=True)
    e = jnp.exp(d - m)
    s = jnp.sum(e, axis=-1, keepdims=True)
    lse_ref[0] = m + jnp.log(s)
    dots_ref[...] = e / s
    for n in range(NBUCKET):
        attn = dots_ref[pl.ds(n * BUCKET, BUCKET), :].astype(jnp.bfloat16)
        so_ref[0, pl.ds(n * BUCKET, BUCKET), :] = lax.dot_general(
            attn, _bk_slice(sv_ref, n), (((1,), (0,)), ((), ())),
            preferred_element_type=jnp.float32).astype(jnp.bfloat16)


def _attn_call(sq, sk, sv):
    return pl.pallas_call(
        _attn_body,
        grid=(B * NH,),
        in_specs=[pl.BlockSpec((1, T, D), lambda i: (i, 0, 0))] * 3,
        out_specs=[
            pl.BlockSpec((1, T, D), lambda i: (i, 0, 0)),
            pl.BlockSpec((1, T, 1), lambda i: (i, 0, 0)),
        ],
        out_shape=[
            jax.ShapeDtypeStruct((B * NH, T, D), jnp.bfloat16),
            jax.ShapeDtypeStruct((B * NH, T, 1), jnp.float32),
        ],
        scratch_shapes=[pltpu.VMEM((T, D), jnp.float32)],
    )(sq, sk, sv)


_NC, _NS = 2, 16
_NW = _NC * _NS
_ROWS = B * NH * T
_RPW = _ROWS // _NW
_CHUNK = 256
_NCHUNK = _RPW // _CHUNK

_SC_MESH = plsc.VectorSubcoreMesh(core_axis_name="c", subcore_axis_name="s",
                                  num_cores=_NC, num_subcores=_NS)


_DW = D // 2


def _gather3_body(q_hbm, k_hbm, v_hbm, iq_hbm, ik_hbm,
                  sq_hbm, sk_hbm, sv_hbm,
                  iq_v, ik_v, buf0, buf1, sem0, sem1):
    wid = lax.axis_index("s") * _NC + lax.axis_index("c")
    base = wid * _RPW
    pltpu.sync_copy(iq_hbm.at[pl.ds(base, _RPW)], iq_v)
    pltpu.sync_copy(ik_hbm.at[pl.ds(base, _RPW)], ik_v)
    bufs = (buf0, buf1)
    sems = (sem0, sem1)
    for table, idx_v, out in ((q_hbm, iq_v, sq_hbm),
                              (k_hbm, ik_v, sk_hbm),
                              (v_hbm, ik_v, sv_hbm)):
        descs = [None, None]
        descs[0] = pltpu.async_copy(
            table.at[idx_v.at[pl.ds(0, _CHUNK)]], bufs[0], sems[0])
        for c in range(_NCHUNK):
            cur = c % 2
            if c + 1 < _NCHUNK:
                descs[1 - cur] = pltpu.async_copy(
                    table.at[idx_v.at[pl.ds((c + 1) * _CHUNK, _CHUNK)]],
                    bufs[1 - cur], sems[1 - cur])
            descs[cur].wait()
            pltpu.sync_copy(bufs[cur], out.at[pl.ds(base + c * _CHUNK, _CHUNK)])


@functools.partial(
    pl.kernel,
    out_type=[jax.ShapeDtypeStruct((_ROWS, _DW), jnp.float32)] * 3,
    mesh=_SC_MESH,
    compiler_params=pltpu.CompilerParams(use_tc_tiling_on_sc=False),
    scratch_types=[
        pltpu.VMEM((_RPW,), jnp.int32),
        pltpu.VMEM((_RPW,), jnp.int32),
        pltpu.VMEM((_CHUNK, _DW), jnp.float32),
        pltpu.VMEM((_CHUNK, _DW), jnp.float32),
        pltpu.SemaphoreType.DMA,
        pltpu.SemaphoreType.DMA,
    ],
)
def _gather3(q_hbm, k_hbm, v_hbm, iq_hbm, ik_hbm,
             sq_hbm, sk_hbm, sv_hbm, *scratch):
    _gather3_body(q_hbm, k_hbm, v_hbm, iq_hbm, ik_hbm,
                  sq_hbm, sk_hbm, sv_hbm, *scratch)


_NROWS = 2 * B * NH
_ROWS_PER_W = _NROWS // _NW
_NVALS = 2 * NPROJ


def _sort_body(bkt_hbm, stick_hbm, undo_hbm,
               bkt_v, rankg_v, stick_v, undo_v, running, offsets, s16):
    iota = lax.iota(jnp.int32, 16)
    iota_m1 = jnp.maximum(iota - 1, 0)
    iota_p1 = jnp.minimum(iota + 1, 15)
    wid = lax.axis_index("s") * _NC + lax.axis_index("c")

    def one_row(g):
        base = ((g % (B * NH)) // NH) * T
        pltpu.sync_copy(bkt_hbm.at[g], bkt_v)
        for c in range(_NVALS // 16):
            running[pl.ds(c * 16, 16)] = jnp.zeros((16,), jnp.int32)

        def pass1(i, _):
            t0 = i * 16
            b16 = bkt_v[pl.ds(t0, 16)]
            key = b16 * T + (t0 + iota)
            ks = jnp.sort(key)
            bs = jnp.right_shift(ks, 11)
            ts = jnp.bitwise_and(ks, T - 1)
            s16[...] = bs
            pb = plsc.load_gather(s16, [iota_m1])
            nb = plsc.load_gather(s16, [iota_p1])
            change = jnp.logical_or(iota == 0, pb != bs)
            is_last = jnp.logical_or(iota == 15, nb != bs)
            run_start = plsc.cummax(jnp.where(change, iota, 0))
            runn = plsc.load_gather(running, [bs])
            pos = runn + (iota - run_start)
            plsc.store_scatter(running, [bs], pos + 1, mask=is_last)
            plsc.store_scatter(rankg_v, [ts], pos)
            return 0

        lax.fori_loop(0, T // 16, pass1, 0)

        carry = jnp.int32(0)
        for c in range(_NVALS // 16):
            seg = running[pl.ds(c * 16, 16)]
            inc = plsc.cumsum(seg)
            offsets[pl.ds(c * 16, 16)] = inc - seg + carry
            carry = carry + jnp.sum(seg)

        gbase = g * T

        def pass2(i, _):
            t0 = i * 16
            b16 = bkt_v[pl.ds(t0, 16)]
            rg = rankg_v[pl.ds(t0, 16)]
            pos = plsc.load_gather(offsets, [b16]) + rg
            plsc.store_scatter(stick_v, [pos], base + t0 + iota)
            undo_v[pl.ds(t0, 16)] = pos + gbase
            return 0

        lax.fori_loop(0, T // 16, pass2, 0)
        pltpu.sync_copy(stick_v, stick_hbm.at[g])

        @pl.when(g < B * NH)
        def _():
            pltpu.sync_copy(undo_v, undo_hbm.at[g])

    for ri in range(_ROWS_PER_W):
        one_row(wid * _ROWS_PER_W + ri)


@functools.partial(
    pl.kernel,
    out_type=[
        jax.ShapeDtypeStruct((_NROWS, T), jnp.int32),
        jax.ShapeDtypeStruct((B * NH, T), jnp.int32),
    ],
    mesh=_SC_MESH,
    compiler_params=pltpu.CompilerParams(needs_layout_passes=False),
    scratch_types=[
        pltpu.VMEM((T,), jnp.int32),
        pltpu.VMEM((T,), jnp.int32),
        pltpu.VMEM((T,), jnp.int32),
        pltpu.VMEM((T,), jnp.int32),
        pltpu.VMEM((_NVALS,), jnp.int32),
        pltpu.VMEM((_NVALS,), jnp.int32),
        pltpu.VMEM((16,), jnp.int32),
    ],
)
def _bucket_sort(bkt_hbm, stick_hbm, undo_hbm, *scratch):
    _sort_body(bkt_hbm, stick_hbm, undo_hbm, *scratch)


_GROWS = B * NH
_GPW = _GROWS // _NW
_OCHUNK = 256


def _unsort_body(so_hbm, lse_hbm, undo_hbm, o_hbm, lg_hbm,
                 undo_v, lse_v, lg_v, buf0, buf1, sem0, sem1):
    wid = lax.axis_index("s") * _NC + lax.axis_index("c")
    bufs, sems = (buf0, buf1), (sem0, sem1)
    for j in range(_GPW):
        g = wid * _GPW + j
        gbase = g * T
        pltpu.sync_copy(undo_hbm.at[g], undo_v)
        pltpu.sync_copy(lse_hbm.at[g], lse_v)

        def logit_pass(i, _):
            pos = undo_v[pl.ds(i * 16, 16)] - gbase
            lg_v[pl.ds(i * 16, 16)] = plsc.load_gather(lse_v, [pos])
            return 0

        lax.fori_loop(0, T // 16, logit_pass, 0)
        pltpu.sync_copy(lg_v, lg_hbm.at[g])

        descs = [None, None]
        descs[0] = pltpu.async_copy(
            so_hbm.at[undo_v.at[pl.ds(0, _OCHUNK)]], bufs[0], sems[0])
        nch = T // _OCHUNK
        for c in range(nch):
            cur = c % 2
            if c + 1 < nch:
                descs[1 - cur] = pltpu.async_copy(
                    so_hbm.at[undo_v.at[pl.ds((c + 1) * _OCHUNK, _OCHUNK)]],
                    bufs[1 - cur], sems[1 - cur])
            descs[cur].wait()
            pltpu.sync_copy(bufs[cur],
                            o_hbm.at[pl.ds(gbase + c * _OCHUNK, _OCHUNK)])


@functools.partial(
    pl.kernel,
    out_type=[
        jax.ShapeDtypeStruct((_GROWS * T, _DW), jnp.float32),
        jax.ShapeDtypeStruct((_GROWS, T), jnp.float32),
    ],
    mesh=_SC_MESH,
    compiler_params=pltpu.CompilerParams(needs_layout_passes=False,
                                         use_tc_tiling_on_sc=False),
    scratch_types=[
        pltpu.VMEM((T,), jnp.int32),
        pltpu.VMEM((T,), jnp.float32),
        pltpu.VMEM((T,), jnp.float32),
        pltpu.VMEM((_OCHUNK, _DW), jnp.float32),
        pltpu.VMEM((_OCHUNK, _DW), jnp.float32),
        pltpu.SemaphoreType.DMA,
        pltpu.SemaphoreType.DMA,
    ],
)
def _unsort(so_hbm, lse_hbm, undo_hbm, o_hbm, lg_hbm, *scratch):
    _unsort_body(so_hbm, lse_hbm, undo_hbm, o_hbm, lg_hbm, *scratch)


_CT = 256


def _combine_body(o_ref, lg_ref, out_ref):
    l = lg_ref[0]
    m = jnp.max(l, axis=1, keepdims=True)
    e = jnp.exp(l - m)
    w = e / jnp.sum(e, axis=1, keepdims=True)
    acc = o_ref[0, 0] * w[:, 0:1]
    for r in range(1, NH):
        acc = acc + o_ref[0, r] * w[:, r:r + 1]
    out_ref[0] = acc


def _combine_call(o_uns, lg_t):
    return pl.pallas_call(
        _combine_body,
        grid=(B, T // _CT),
        in_specs=[
            pl.BlockSpec((1, NH, _CT, D), lambda i, j: (i, 0, j, 0)),
            pl.BlockSpec((1, _CT, NH), lambda i, j: (i, j, 0)),
        ],
        out_specs=pl.BlockSpec((1, _CT, D), lambda i, j: (i, j, 0)),
        out_shape=jax.ShapeDtypeStruct((B, T, D), jnp.float32),
    )(o_uns, lg_t)


def kernel(q, k, v, proj):
    proj2 = proj.reshape(D, NH * NPROJ)
    x2 = jnp.concatenate([q, k], axis=0)
    buckets = _hash_call(x2, proj2)
    buckets = buckets.transpose(0, 2, 1)

    stick_abs, undo = _bucket_sort(buckets.reshape(_NROWS, T))

    iq = stick_abs[:B * NH].reshape(-1)
    ik = stick_abs[B * NH:].reshape(-1)

    def _to_words(x):
        xb = x.astype(jnp.bfloat16).reshape(B * T, _DW, 2)
        return lax.bitcast_convert_type(xb, jnp.float32)

    def _to_bf16(xw, shape):
        return lax.bitcast_convert_type(xw, jnp.bfloat16).reshape(shape)

    sqw, skw, svw = _gather3(_to_words(q), _to_words(k), _to_words(v), iq, ik)

    so, slse = _attn_call(_to_bf16(sqw, (B * NH, T, D)),
                          _to_bf16(skw, (B * NH, T, D)),
                          _to_bf16(svw, (B * NH, T, D)))

    sow = lax.bitcast_convert_type(so.reshape(B * NH * T, _DW, 2),
                                   jnp.float32)
    ow_uns, lg_uns = _unsort(sow, slse.reshape(B * NH, T), undo)
    lg_t = lg_uns.reshape(B, NH, T).transpose(0, 2, 1)
    return _combine_call(_to_bf16(ow_uns, (B, NH, T, D)), lg_t)


# speedup vs baseline: 2.0063x; 2.0063x over previous
import functools

import jax
import jax.numpy as jnp
from jax import lax
from jax.experimental import pallas as pl
from jax.experimental.pallas import tpu as pltpu
from jax.experimental.pallas import tpu_sc as plsc

B, T, D = 16, 2048, 128
NH = 8
NPROJ = 32
BUCKET = 64
NBUCKET = T // BUCKET
SCALE = D ** -0.5


def _hash_body(x_ref, proj_ref, out_ref):
    x = x_ref[0]
    p = proj_ref[...]
    px = lax.dot_general(x, p, (((1,), (0,)), ((), ())),
                         preferred_element_type=jnp.float32)
    px3 = px.reshape(T, NH, NPROJ)
    m = jnp.max(jnp.abs(px3), axis=-1, keepdims=True)
    iota = lax.broadcasted_iota(jnp.int32, (T, NH, NPROJ), 2)
    big = jnp.int32(2 * NPROJ)
    cand = jnp.minimum(jnp.where(px3 >= m, iota, big),
                       jnp.where(px3 <= -m, iota + NPROJ, big))
    out_ref[0] = jnp.min(cand, axis=-1)


def _hash_call(x2, proj2):
    return pl.pallas_call(
        _hash_body,
        grid=(2 * B,),
        in_specs=[
            pl.BlockSpec((1, T, D), lambda i: (i, 0, 0)),
            pl.BlockSpec((D, NH * NPROJ), lambda i: (0, 0)),
        ],
        out_specs=pl.BlockSpec((1, T, NH), lambda i: (i, 0, 0)),
        out_shape=jax.ShapeDtypeStruct((2 * B, T, NH), jnp.int32),
    )(x2, proj2)


def _halo_slice(ref, n, c0):
    lo = (n - 1) % NBUCKET
    if n == 0:
        return jnp.concatenate(
            [ref[0, pl.ds(lo * BUCKET, BUCKET), pl.ds(c0, D)],
             ref[0, pl.ds(0, BUCKET), pl.ds(c0, D)]], axis=0)
    return ref[0, pl.ds(lo * BUCKET, 2 * BUCKET), pl.ds(c0, D)]


def _attn_body(sq_ref, skv_ref, so_ref, lse_ref, dots_ref):
    for n in range(NBUCKET):
        bq = sq_ref[0, pl.ds(n * BUCKET, BUCKET), :].astype(jnp.bfloat16)
        dots_ref[pl.ds(n * BUCKET, BUCKET), :] = lax.dot_general(
            bq, _halo_slice(skv_ref, n, 0), (((1,), (1,)), ((), ())),
            preferred_element_type=jnp.float32)
    d = dots_ref[...] * SCALE
    m = jnp.max(d, axis=-1, keepdims=True)
    e = jnp.exp(d - m)
    s = jnp.sum(e, axis=-1, keepdims=True)
    lse_ref[0] = m + jnp.log(s)
    dots_ref[...] = e / s
    for n in range(NBUCKET):
        attn = dots_ref[pl.ds(n * BUCKET, BUCKET), :].astype(jnp.bfloat16)
        so_ref[0, pl.ds(n * BUCKET, BUCKET), :] = lax.dot_general(
            attn, _halo_slice(skv_ref, n, D), (((1,), (0,)), ((), ())),
            preferred_element_type=jnp.float32)


def _attn_call(sq, skv):
    return pl.pallas_call(
        _attn_body,
        grid=(B * NH,),
        in_specs=[
            pl.BlockSpec((1, T, D), lambda i: (i, 0, 0)),
            pl.BlockSpec((1, T, 2 * D), lambda i: (i, 0, 0)),
        ],
        out_specs=[
            pl.BlockSpec((1, T, D), lambda i: (i, 0, 0)),
            pl.BlockSpec((1, T, 1), lambda i: (i, 0, 0)),
        ],
        out_shape=[
            jax.ShapeDtypeStruct((B * NH, T, D), jnp.float32),
            jax.ShapeDtypeStruct((B * NH, T, 1), jnp.float32),
        ],
        scratch_shapes=[pltpu.VMEM((T, D), jnp.float32)],
    )(sq, skv)


_NC, _NS = 2, 16
_NW = _NC * _NS
_ROWS = B * NH * T
_RPW = _ROWS // _NW
_CHUNK = 256
_NCHUNK = _RPW // _CHUNK

_SC_MESH = plsc.VectorSubcoreMesh(core_axis_name="c", subcore_axis_name="s",
                                  num_cores=_NC, num_subcores=_NS)


_DW = D // 2


def _gather2_body(q_hbm, kv_hbm, iq_hbm, ik_hbm,
                  sq_hbm, skv_hbm,
                  iq_v, ik_v, buf0, buf1, sem0, sem1):
    wid = lax.axis_index("s") * _NC + lax.axis_index("c")
    base = wid * _RPW
    pltpu.sync_copy(iq_hbm.at[pl.ds(base, _RPW)], iq_v)
    pltpu.sync_copy(ik_hbm.at[pl.ds(base, _RPW)], ik_v)
    bufs = (buf0, buf1)
    sems = (sem0, sem1)
    for table, idx_v, out in ((q_hbm, iq_v, sq_hbm),
                              (kv_hbm, ik_v, skv_hbm)):
        descs = [None, None]
        descs[0] = pltpu.async_copy(
            table.at[idx_v.at[pl.ds(0, _CHUNK)]], bufs[0], sems[0])
        for c in range(_NCHUNK):
            cur = c % 2
            if c + 1 < _NCHUNK:
                descs[1 - cur] = pltpu.async_copy(
                    table.at[idx_v.at[pl.ds((c + 1) * _CHUNK, _CHUNK)]],
                    bufs[1 - cur], sems[1 - cur])
            descs[cur].wait()
            pltpu.sync_copy(bufs[cur], out.at[pl.ds(base + c * _CHUNK, _CHUNK)])


@functools.partial(
    pl.kernel,
    out_type=[jax.ShapeDtypeStruct((_ROWS, D), jnp.float32)] * 2,
    mesh=_SC_MESH,
    scratch_types=[
        pltpu.VMEM((_RPW,), jnp.int32),
        pltpu.VMEM((_RPW,), jnp.int32),
        pltpu.VMEM((_CHUNK, D), jnp.float32),
        pltpu.VMEM((_CHUNK, D), jnp.float32),
        pltpu.SemaphoreType.DMA,
        pltpu.SemaphoreType.DMA,
    ],
)
def _gather2(q_hbm, kv_hbm, iq_hbm, ik_hbm, sq_hbm, skv_hbm, *scratch):
    _gather2_body(q_hbm, kv_hbm, iq_hbm, ik_hbm, sq_hbm, skv_hbm, *scratch)


_NROWS = 2 * B * NH
_ROWS_PER_W = _NROWS // _NW
_NVALS = 2 * NPROJ


def _sort_body(bkt_hbm, stick_hbm, undo_hbm,
               bkt_v, rankg_v, stick_v, undo_v, running, offsets, s16):
    iota = lax.iota(jnp.int32, 16)
    iota_m1 = jnp.maximum(iota - 1, 0)
    iota_p1 = jnp.minimum(iota + 1, 15)
    wid = lax.axis_index("s") * _NC + lax.axis_index("c")

    def one_row(g):
        base = ((g % (B * NH)) // NH) * T
        pltpu.sync_copy(bkt_hbm.at[g], bkt_v)
        for c in range(_NVALS // 16):
            running[pl.ds(c * 16, 16)] = jnp.zeros((16,), jnp.int32)

        def pass1(i, _):
            t0 = i * 16
            b16 = bkt_v[pl.ds(t0, 16)]
            key = b16 * T + (t0 + iota)
            ks = jnp.sort(key)
            bs = jnp.right_shift(ks, 11)
            ts = jnp.bitwise_and(ks, T - 1)
            s16[...] = bs
            pb = plsc.load_gather(s16, [iota_m1])
            nb = plsc.load_gather(s16, [iota_p1])
            change = jnp.logical_or(iota == 0, pb != bs)
            is_last = jnp.logical_or(iota == 15, nb != bs)
            run_start = plsc.cummax(jnp.where(change, iota, 0))
            runn = plsc.load_gather(running, [bs])
            pos = runn + (iota - run_start)
            plsc.store_scatter(running, [bs], pos + 1, mask=is_last)
            plsc.store_scatter(rankg_v, [ts], pos)
            return 0

        lax.fori_loop(0, T // 16, pass1, 0)

        carry = jnp.int32(0)
        for c in range(_NVALS // 16):
            seg = running[pl.ds(c * 16, 16)]
            inc = plsc.cumsum(seg)
            offsets[pl.ds(c * 16, 16)] = inc - seg + carry
            carry = carry + jnp.sum(seg)

        gbase = g * T

        def pass2(i, _):
            t0 = i * 16
            b16 = bkt_v[pl.ds(t0, 16)]
            rg = rankg_v[pl.ds(t0, 16)]
            pos = plsc.load_gather(offsets, [b16]) + rg
            plsc.store_scatter(stick_v, [pos], base + t0 + iota)
            undo_v[pl.ds(t0, 16)] = pos + gbase
            return 0

        lax.fori_loop(0, T // 16, pass2, 0)
        pltpu.sync_copy(stick_v, stick_hbm.at[g])

        @pl.when(g < B * NH)
        def _():
            pltpu.sync_copy(undo_v, undo_hbm.at[g])

    for ri in range(_ROWS_PER_W):
        one_row(wid * _ROWS_PER_W + ri)


@functools.partial(
    pl.kernel,
    out_type=[
        jax.ShapeDtypeStruct((_NROWS, T), jnp.int32),
        jax.ShapeDtypeStruct((B * NH, T), jnp.int32),
    ],
    mesh=_SC_MESH,
    compiler_params=pltpu.CompilerParams(needs_layout_passes=False),
    scratch_types=[
        pltpu.VMEM((T,), jnp.int32),
        pltpu.VMEM((T,), jnp.int32),
        pltpu.VMEM((T,), jnp.int32),
        pltpu.VMEM((T,), jnp.int32),
        pltpu.VMEM((_NVALS,), jnp.int32),
        pltpu.VMEM((_NVALS,), jnp.int32),
        pltpu.VMEM((16,), jnp.int32),
    ],
)
def _bucket_sort(bkt_hbm, stick_hbm, undo_hbm, *scratch):
    _sort_body(bkt_hbm, stick_hbm, undo_hbm, *scratch)


_GROWS = B * NH
_GPW = _GROWS // _NW
_OCHUNK = 256


def _unsort_body(so_hbm, lse_hbm, undo_hbm, o_hbm, lg_hbm,
                 undo_v, lse_v, lg_v, buf0, buf1, sem0, sem1):
    wid = lax.axis_index("s") * _NC + lax.axis_index("c")
    bufs, sems = (buf0, buf1), (sem0, sem1)
    for j in range(_GPW):
        g = wid * _GPW + j
        gbase = g * T
        pltpu.sync_copy(undo_hbm.at[g], undo_v)
        pltpu.sync_copy(lse_hbm.at[g], lse_v)

        def logit_pass(i, _):
            pos = undo_v[pl.ds(i * 16, 16)] - gbase
            lg_v[pl.ds(i * 16, 16)] = plsc.load_gather(lse_v, [pos])
            return 0

        lax.fori_loop(0, T // 16, logit_pass, 0)
        pltpu.sync_copy(lg_v, lg_hbm.at[g])

        descs = [None, None]
        descs[0] = pltpu.async_copy(
            so_hbm.at[undo_v.at[pl.ds(0, _OCHUNK)]], bufs[0], sems[0])
        nch = T // _OCHUNK
        for c in range(nch):
            cur = c % 2
            if c + 1 < nch:
                descs[1 - cur] = pltpu.async_copy(
                    so_hbm.at[undo_v.at[pl.ds((c + 1) * _OCHUNK, _OCHUNK)]],
                    bufs[1 - cur], sems[1 - cur])
            descs[cur].wait()
            pltpu.sync_copy(bufs[cur],
                            o_hbm.at[pl.ds(gbase + c * _OCHUNK, _OCHUNK)])


@functools.partial(
    pl.kernel,
    out_type=[
        jax.ShapeDtypeStruct((_GROWS * T, D), jnp.float32),
        jax.ShapeDtypeStruct((_GROWS, T), jnp.float32),
    ],
    mesh=_SC_MESH,
    compiler_params=pltpu.CompilerParams(needs_layout_passes=False),
    scratch_types=[
        pltpu.VMEM((T,), jnp.int32),
        pltpu.VMEM((T,), jnp.float32),
        pltpu.VMEM((T,), jnp.float32),
        pltpu.VMEM((_OCHUNK, D), jnp.float32),
        pltpu.VMEM((_OCHUNK, D), jnp.float32),
        pltpu.SemaphoreType.DMA,
        pltpu.SemaphoreType.DMA,
    ],
)
def _unsort(so_hbm, lse_hbm, undo_hbm, o_hbm, lg_hbm, *scratch):
    _unsort_body(so_hbm, lse_hbm, undo_hbm, o_hbm, lg_hbm, *scratch)


_CT = 256


def _combine_body(o_ref, lg_ref, out_ref):
    l = lg_ref[0]
    m = jnp.max(l, axis=1, keepdims=True)
    e = jnp.exp(l - m)
    w = e / jnp.sum(e, axis=1, keepdims=True)
    acc = o_ref[0, 0] * w[:, 0:1]
    for r in range(1, NH):
        acc = acc + o_ref[0, r] * w[:, r:r + 1]
    out_ref[0] = acc


def _combine_call(o_uns, lg_t):
    return pl.pallas_call(
        _combine_body,
        grid=(B, T // _CT),
        in_specs=[
            pl.BlockSpec((1, NH, _CT, D), lambda i, j: (i, 0, j, 0)),
            pl.BlockSpec((1, _CT, NH), lambda i, j: (i, j, 0)),
        ],
        out_specs=pl.BlockSpec((1, _CT, D), lambda i, j: (i, j, 0)),
        out_shape=jax.ShapeDtypeStruct((B, T, D), jnp.float32),
    )(o_uns, lg_t)


def kernel(q, k, v, proj):
    proj2 = proj.reshape(D, NH * NPROJ)
    x2 = jnp.concatenate([q, k], axis=0)
    buckets = _hash_call(x2, proj2)
    buckets = buckets.transpose(0, 2, 1)

    stick_abs, undo = _bucket_sort(buckets.reshape(_NROWS, T))

    iq = stick_abs[:B * NH].reshape(-1)
    ik = stick_abs[B * NH:].reshape(-1)

    def _to_words(x):
        xb = x.astype(jnp.bfloat16).reshape(B * T, _DW, 2)
        return lax.bitcast_convert_type(xb, jnp.float32)

    kvw = jnp.concatenate([_to_words(k), _to_words(v)], axis=1)
    sq, skvw = _gather2(q.reshape(B * T, D), kvw, iq, ik)

    skv = lax.bitcast_convert_type(
        skvw, jnp.bfloat16).reshape(B * NH, T, 2 * D)

    so, slse = _attn_call(sq.reshape(B * NH, T, D), skv)

    o_uns, lg_uns = _unsort(so.reshape(B * NH * T, D),
                            slse.reshape(B * NH, T), undo)
    lg_t = lg_uns.reshape(B, NH, T).transpose(0, 2, 1)
    return _combine_call(o_uns.reshape(B, NH, T, D), lg_t)


# speedup vs baseline: 4.0550x; 2.0211x over previous
import functools

import jax
import jax.numpy as jnp
from jax import lax
from jax.experimental import pallas as pl
from jax.experimental.pallas import tpu as pltpu
from jax.experimental.pallas import tpu_sc as plsc

B, T, D = 16, 2048, 128
NH = 8
NPROJ = 32
BUCKET = 64
NBUCKET = T // BUCKET
SCALE = D ** -0.5


def _hash_body(x_ref, proj_ref, out_ref):
    x = x_ref[0]
    p = proj_ref[...]
    px = lax.dot_general(x, p, (((1,), (0,)), ((), ())),
                         preferred_element_type=jnp.float32)
    px3 = px.reshape(T, NH, NPROJ)
    m = jnp.max(jnp.abs(px3), axis=-1, keepdims=True)
    iota = lax.broadcasted_iota(jnp.int32, (T, NH, NPROJ), 2)
    big = jnp.int32(2 * NPROJ)
    cand = jnp.minimum(jnp.where(px3 >= m, iota, big),
                       jnp.where(px3 <= -m, iota + NPROJ, big))
    out_ref[0] = jnp.min(cand, axis=-1)


def _hash_call(x2, proj2):
    return pl.pallas_call(
        _hash_body,
        grid=(2 * B,),
        in_specs=[
            pl.BlockSpec((1, T, D), lambda i: (i, 0, 0)),
            pl.BlockSpec((D, NH * NPROJ), lambda i: (0, 0)),
        ],
        out_specs=pl.BlockSpec((1, T, NH), lambda i: (i, 0, 0)),
        out_shape=jax.ShapeDtypeStruct((2 * B, T, NH), jnp.int32),
    )(x2, proj2)


def _bk_slice(ref, n):
    lo = (n - 1) % NBUCKET
    if n == 0:
        return jnp.concatenate(
            [ref[0, pl.ds(lo * BUCKET, BUCKET), :],
             ref[0, pl.ds(0, BUCKET), :]], axis=0)
    return ref[0, pl.ds(lo * BUCKET, 2 * BUCKET), :]


def _attn_body(sq_ref, sk_ref, sv_ref, so_ref, lse_ref, dots_ref):
    for n in range(NBUCKET):
        bq = sq_ref[0, pl.ds(n * BUCKET, BUCKET), :]
        dots_ref[pl.ds(n * BUCKET, BUCKET), :] = lax.dot_general(
            bq, _bk_slice(sk_ref, n), (((1,), (1,)), ((), ())),
            preferred_element_type=jnp.float32)
    d = dots_ref[...] * SCALE
    m = jnp.max(d, axis=-1, keepdims=True)
    e = jnp.exp(d - m)
    s = jnp.sum(e, axis=-1, keepdims=True)
    lse_ref[0] = m + jnp.log(s)
    dots_ref[...] = e / s
    for n in range(NBUCKET):
        attn = dots_ref[pl.ds(n * BUCKET, BUCKET), :]
        so_ref[0, pl.ds(n * BUCKET, BUCKET), :] = lax.dot_general(
            attn, _bk_slice(sv_ref, n), (((1,), (0,)), ((), ())),
            preferred_element_type=jnp.float32)


def _attn_call(sq, sk, sv):
    return pl.pallas_call(
        _attn_body,
        grid=(B * NH,),
        in_specs=[pl.BlockSpec((1, T, D), lambda i: (i, 0, 0))] * 3,
        out_specs=[
            pl.BlockSpec((1, T, D), lambda i: (i, 0, 0)),
            pl.BlockSpec((1, T, 1), lambda i: (i, 0, 0)),
        ],
        out_shape=[
            jax.ShapeDtypeStruct((B * NH, T, D), jnp.float32),
            jax.ShapeDtypeStruct((B * NH, T, 1), jnp.float32),
        ],
        scratch_shapes=[pltpu.VMEM((T, D), jnp.float32)],
    )(sq, sk, sv)


_NC, _NS = 2, 16
_NW = _NC * _NS
_ROWS = B * NH * T
_RPW = _ROWS // _NW
_CHUNK = 256
_NCHUNK = _RPW // _CHUNK

_SC_MESH = plsc.VectorSubcoreMesh(core_axis_name="c", subcore_axis_name="s",
                                  num_cores=_NC, num_subcores=_NS)


def _gather3_body(q_hbm, k_hbm, v_hbm, iq_hbm, ik_hbm,
                  sq_hbm, sk_hbm, sv_hbm,
                  iq_v, ik_v, buf0, buf1, sem0, sem1):
    wid = lax.axis_index("s") * _NC + lax.axis_index("c")
    base = wid * _RPW
    pltpu.sync_copy(iq_hbm.at[pl.ds(base, _RPW)], iq_v)
    pltpu.sync_copy(ik_hbm.at[pl.ds(base, _RPW)], ik_v)
    bufs = (buf0, buf1)
    sems = (sem0, sem1)
    for table, idx_v, out in ((q_hbm, iq_v, sq_hbm),
                              (k_hbm, ik_v, sk_hbm),
                              (v_hbm, ik_v, sv_hbm)):
        descs = [None, None]
        descs[0] = pltpu.async_copy(
            table.at[idx_v.at[pl.ds(0, _CHUNK)]], bufs[0], sems[0])
        for c in range(_NCHUNK):
            cur = c % 2
            if c + 1 < _NCHUNK:
                descs[1 - cur] = pltpu.async_copy(
                    table.at[idx_v.at[pl.ds((c + 1) * _CHUNK, _CHUNK)]],
                    bufs[1 - cur], sems[1 - cur])
            descs[cur].wait()
            pltpu.sync_copy(bufs[cur], out.at[pl.ds(base + c * _CHUNK, _CHUNK)])


@functools.partial(
    pl.kernel,
    out_type=[jax.ShapeDtypeStruct((_ROWS, D), jnp.float32)] * 3,
    mesh=_SC_MESH,
    scratch_types=[
        pltpu.VMEM((_RPW,), jnp.int32),
        pltpu.VMEM((_RPW,), jnp.int32),
        pltpu.VMEM((_CHUNK, D), jnp.float32),
        pltpu.VMEM((_CHUNK, D), jnp.float32),
        pltpu.SemaphoreType.DMA,
        pltpu.SemaphoreType.DMA,
    ],
)
def _gather3(q_hbm, k_hbm, v_hbm, iq_hbm, ik_hbm,
             sq_hbm, sk_hbm, sv_hbm, *scratch):
    _gather3_body(q_hbm, k_hbm, v_hbm, iq_hbm, ik_hbm,
                  sq_hbm, sk_hbm, sv_hbm, *scratch)


_NROWS = 2 * B * NH
_ROWS_PER_W = _NROWS // _NW
_NVALS = 2 * NPROJ


def _sort_body(bkt_hbm, stick_hbm, undo_hbm,
               bkt_v, rankg_v, stick_v, undo_v, running, offsets, s16):
    iota = lax.iota(jnp.int32, 16)
    iota_m1 = jnp.maximum(iota - 1, 0)
    iota_p1 = jnp.minimum(iota + 1, 15)
    wid = lax.axis_index("s") * _NC + lax.axis_index("c")

    def one_row(g):
        base = ((g % (B * NH)) // NH) * T
        pltpu.sync_copy(bkt_hbm.at[g], bkt_v)
        for c in range(_NVALS // 16):
            running[pl.ds(c * 16, 16)] = jnp.zeros((16,), jnp.int32)

        def pass1(i, _):
            t0 = i * 16
            b16 = bkt_v[pl.ds(t0, 16)]
            key = b16 * T + (t0 + iota)
            ks = jnp.sort(key)
            bs = jnp.right_shift(ks, 11)
            ts = jnp.bitwise_and(ks, T - 1)
            s16[...] = bs
            pb = plsc.load_gather(s16, [iota_m1])
            nb = plsc.load_gather(s16, [iota_p1])
            change = jnp.logical_or(iota == 0, pb != bs)
            is_last = jnp.logical_or(iota == 15, nb != bs)
            run_start = plsc.cummax(jnp.where(change, iota, 0))
            runn = plsc.load_gather(running, [bs])
            pos = runn + (iota - run_start)
            plsc.store_scatter(running, [bs], pos + 1, mask=is_last)
            plsc.store_scatter(rankg_v, [ts], pos)
            return 0

        lax.fori_loop(0, T // 16, pass1, 0)

        carry = jnp.int32(0)
        for c in range(_NVALS // 16):
            seg = running[pl.ds(c * 16, 16)]
            inc = plsc.cumsum(seg)
            offsets[pl.ds(c * 16, 16)] = inc - seg + carry
            carry = carry + jnp.sum(seg)

        gbase = g * T

        def pass2(i, _):
            t0 = i * 16
            b16 = bkt_v[pl.ds(t0, 16)]
            rg = rankg_v[pl.ds(t0, 16)]
            pos = plsc.load_gather(offsets, [b16]) + rg
            plsc.store_scatter(stick_v, [pos], base + t0 + iota)
            undo_v[pl.ds(t0, 16)] = pos + gbase
            return 0

        lax.fori_loop(0, T // 16, pass2, 0)
        pltpu.sync_copy(stick_v, stick_hbm.at[g])

        @pl.when(g < B * NH)
        def _():
            pltpu.sync_copy(undo_v, undo_hbm.at[g])

    for ri in range(_ROWS_PER_W):
        one_row(wid * _ROWS_PER_W + ri)


@functools.partial(
    pl.kernel,
    out_type=[
        jax.ShapeDtypeStruct((_NROWS, T), jnp.int32),
        jax.ShapeDtypeStruct((B * NH, T), jnp.int32),
    ],
    mesh=_SC_MESH,
    compiler_params=pltpu.CompilerParams(needs_layout_passes=False),
    scratch_types=[
        pltpu.VMEM((T,), jnp.int32),
        pltpu.VMEM((T,), jnp.int32),
        pltpu.VMEM((T,), jnp.int32),
        pltpu.VMEM((T,), jnp.int32),
        pltpu.VMEM((_NVALS,), jnp.int32),
        pltpu.VMEM((_NVALS,), jnp.int32),
        pltpu.VMEM((16,), jnp.int32),
    ],
)
def _bucket_sort(bkt_hbm, stick_hbm, undo_hbm, *scratch):
    _sort_body(bkt_hbm, stick_hbm, undo_hbm, *scratch)


_GROWS = B * NH
_GPW = _GROWS // _NW
_OCHUNK = 256


def _unsort_body(so_hbm, lse_hbm, undo_hbm, o_hbm, lg_hbm,
                 undo_v, lse_v, lg_v, buf0, buf1, sem0, sem1):
    wid = lax.axis_index("s") * _NC + lax.axis_index("c")
    bufs, sems = (buf0, buf1), (sem0, sem1)
    for j in range(_GPW):
        g = wid * _GPW + j
        gbase = g * T
        pltpu.sync_copy(undo_hbm.at[g], undo_v)
        pltpu.sync_copy(lse_hbm.at[g], lse_v)

        def logit_pass(i, _):
            pos = undo_v[pl.ds(i * 16, 16)] - gbase
            lg_v[pl.ds(i * 16, 16)] = plsc.load_gather(lse_v, [pos])
            return 0

        lax.fori_loop(0, T // 16, logit_pass, 0)
        pltpu.sync_copy(lg_v, lg_hbm.at[g])

        descs = [None, None]
        descs[0] = pltpu.async_copy(
            so_hbm.at[undo_v.at[pl.ds(0, _OCHUNK)]], bufs[0], sems[0])
        nch = T // _OCHUNK
        for c in range(nch):
            cur = c % 2
            if c + 1 < nch:
                descs[1 - cur] = pltpu.async_copy(
                    so_hbm.at[undo_v.at[pl.ds((c + 1) * _OCHUNK, _OCHUNK)]],
                    bufs[1 - cur], sems[1 - cur])
            descs[cur].wait()
            pltpu.sync_copy(bufs[cur],
                            o_hbm.at[pl.ds(gbase + c * _OCHUNK, _OCHUNK)])


@functools.partial(
    pl.kernel,
    out_type=[
        jax.ShapeDtypeStruct((_GROWS * T, D), jnp.float32),
        jax.ShapeDtypeStruct((_GROWS, T), jnp.float32),
    ],
    mesh=_SC_MESH,
    compiler_params=pltpu.CompilerParams(needs_layout_passes=False),
    scratch_types=[
        pltpu.VMEM((T,), jnp.int32),
        pltpu.VMEM((T,), jnp.float32),
        pltpu.VMEM((T,), jnp.float32),
        pltpu.VMEM((_OCHUNK, D), jnp.float32),
        pltpu.VMEM((_OCHUNK, D), jnp.float32),
        pltpu.SemaphoreType.DMA,
        pltpu.SemaphoreType.DMA,
    ],
)
def _unsort(so_hbm, lse_hbm, undo_hbm, o_hbm, lg_hbm, *scratch):
    _unsort_body(so_hbm, lse_hbm, undo_hbm, o_hbm, lg_hbm, *scratch)


_CT = 256


def _combine_body(o_ref, lg_ref, out_ref):
    l = lg_ref[0]
    m = jnp.max(l, axis=1, keepdims=True)
    e = jnp.exp(l - m)
    w = e / jnp.sum(e, axis=1, keepdims=True)
    acc = o_ref[0, 0] * w[:, 0:1]
    for r in range(1, NH):
        acc = acc + o_ref[0, r] * w[:, r:r + 1]
    out_ref[0] = acc


def _combine_call(o_uns, lg_t):
    return pl.pallas_call(
        _combine_body,
        grid=(B, T // _CT),
        in_specs=[
            pl.BlockSpec((1, NH, _CT, D), lambda i, j: (i, 0, j, 0)),
            pl.BlockSpec((1, _CT, NH), lambda i, j: (i, j, 0)),
        ],
        out_specs=pl.BlockSpec((1, _CT, D), lambda i, j: (i, j, 0)),
        out_shape=jax.ShapeDtypeStruct((B, T, D), jnp.float32),
    )(o_uns, lg_t)


def kernel(q, k, v, proj):
    proj2 = proj.reshape(D, NH * NPROJ)
    x2 = jnp.concatenate([q, k], axis=0)
    buckets = _hash_call(x2, proj2)
    buckets = buckets.transpose(0, 2, 1)

    stick_abs, undo = _bucket_sort(buckets.reshape(_NROWS, T))

    iq = stick_abs[:B * NH].reshape(-1)
    ik = stick_abs[B * NH:].reshape(-1)

    sq, sk, sv = _gather3(q.reshape(B * T, D), k.reshape(B * T, D),
                          v.reshape(B * T, D), iq, ik)

    so, slse = _attn_call(sq.reshape(B * NH, T, D),
                          sk.reshape(B * NH, T, D),
                          sv.reshape(B * NH, T, D))

    o_uns, lg_uns = _unsort(so.reshape(B * NH * T, D),
                            slse.reshape(B * NH, T), undo)
    lg_t = lg_uns.reshape(B, NH, T).transpose(0, 2, 1)
    return _combine_call(o_uns.reshape(B, NH, T, D), lg_t)
